# baseline (speedup 1.0000x reference)
import jax
import jax.numpy as jnp
from jax.experimental import pallas as pl


def _copy_kernel(cell_ref, out_ref):
    out_ref[...] = cell_ref[...]


def kernel(natoms, pred_distance_displace, pred_var_displace,
           pred_distance_relaxed, pred_var_relaxed, pred_cell):
    cell2d = pred_cell.reshape(128, 9)
    cell_out = pl.pallas_call(
        _copy_kernel,
        out_shape=jax.ShapeDtypeStruct((128, 9), jnp.float32),
    )(cell2d)
    return (pred_distance_displace, pred_var_displace,
            pred_distance_relaxed, pred_var_relaxed,
            cell_out.reshape(128, 3, 3))
